# baseline (speedup 1.0000x reference)
import functools

import jax
import jax.numpy as jnp
from jax import lax
from jax.experimental import pallas as pl
from jax.experimental.pallas import tpu as pltpu
from jax.experimental.pallas import tpu_sc as plsc

BATCH = 4
SEQ_LEN = 8192
D_MODEL = 768

NUM_CORES = 2
NUM_SUBCORES = 16
NUM_WORKERS = NUM_CORES * NUM_SUBCORES
ROWS_PER_WORKER = SEQ_LEN // NUM_WORKERS
CHUNK = 64
CHUNKS_PER_WORKER = ROWS_PER_WORKER // CHUNK


def _pe_broadcast_kernel(src_table_hbm, tgt_table_hbm, src_out_hbm,
                         tgt_out_hbm, buf0, buf1, rsem0, rsem1, wsem0,
                         wsem1):
    wid = lax.axis_index("s") * NUM_CORES + lax.axis_index("c")
    base = wid * ROWS_PER_WORKER
    bufs = (buf0, buf1)
    rsems = (rsem0, rsem1)
    wsems = (wsem0, wsem1)

    tasks = []
    for table_hbm, out_hbm in ((src_table_hbm, src_out_hbm),
                               (tgt_table_hbm, tgt_out_hbm)):
        for c in range(CHUNKS_PER_WORKER):
            tasks.append((table_hbm, out_hbm, base + c * CHUNK))

    n = len(tasks)
    reads = [None] * n
    writes = [None] * n
    for c in range(min(2, n)):
        table_hbm, _, start = tasks[c]
        reads[c] = pltpu.make_async_copy(
            table_hbm.at[pl.ds(start, CHUNK)], bufs[c % 2], rsems[c % 2])
        reads[c].start()
    for c in range(n):
        j = c % 2
        if c >= 2:
            for w in writes[c - 2]:
                w.wait()
            table_hbm, _, start = tasks[c]
            reads[c] = pltpu.make_async_copy(
                table_hbm.at[pl.ds(start, CHUNK)], bufs[j], rsems[j])
            reads[c].start()
        reads[c].wait()
        _, out_hbm, start = tasks[c]
        ws = []
        for b in range(BATCH):
            w = pltpu.make_async_copy(
                bufs[j], out_hbm.at[b, pl.ds(start, CHUNK)], wsems[j])
            w.start()
            ws.append(w)
        writes[c] = ws
    for c in (n - 2, n - 1):
        for w in writes[c]:
            w.wait()


@functools.partial(
    pl.kernel,
    out_type=(
        jax.ShapeDtypeStruct((BATCH, SEQ_LEN, D_MODEL), jnp.float32),
        jax.ShapeDtypeStruct((BATCH, SEQ_LEN, D_MODEL), jnp.float32),
    ),
    mesh=plsc.VectorSubcoreMesh(core_axis_name="c", subcore_axis_name="s"),
    scratch_types=[
        pltpu.VMEM((CHUNK, D_MODEL), jnp.float32),
        pltpu.VMEM((CHUNK, D_MODEL), jnp.float32),
        pltpu.SemaphoreType.DMA,
        pltpu.SemaphoreType.DMA,
        pltpu.SemaphoreType.DMA,
        pltpu.SemaphoreType.DMA,
    ],
)
def _pe_broadcast(src_table_hbm, tgt_table_hbm, src_out_hbm, tgt_out_hbm,
                  buf0, buf1, rsem0, rsem1, wsem0, wsem1):
    _pe_broadcast_kernel(src_table_hbm, tgt_table_hbm, src_out_hbm,
                         tgt_out_hbm, buf0, buf1, rsem0, rsem1, wsem0,
                         wsem1)


def kernel(src_sequences, target_sequences, src_table, tgt_table):
    del src_sequences, target_sequences
    return _pe_broadcast(src_table, tgt_table)
